# baseline (speedup 1.0000x reference)
import functools

import jax
import jax.numpy as jnp
from jax.experimental import pallas as pl
from jax.experimental.pallas import tpu as pltpu

_NCLS = 19
_BY = 128
_CH = 16
_PACK = 1024.0


def _part_kernel(x_ref, t_ref, out_ref):
    j = pl.program_id(1)
    nj = pl.num_programs(1)

    @pl.when(j == 0)
    def _init():
        out_ref[...] = jnp.zeros_like(out_ref)

    r = _CH // 8
    for base in range(0, _BY, _CH):
        rows = pl.ds(base, _CH)
        t = t_ref[0, rows, :]

        parts = []
        for k in range(4):
            mk = x_ref[0, k, rows, :]
            for c in range(4 + k, _NCLS, 4):
                mk = jnp.maximum(mk, x_ref[0, c, rows, :])
            parts.append(mk)
        m = jnp.maximum(jnp.maximum(parts[0], parts[1]),
                        jnp.maximum(parts[2], parts[3]))

        sums = []
        for k in range(4):
            sk = jnp.exp(x_ref[0, k, rows, :] - m)
            for c in range(4 + k, _NCLS, 4):
                sk = sk + jnp.exp(x_ref[0, c, rows, :] - m)
            sums.append(sk)
        s = (sums[0] + sums[1]) + (sums[2] + sums[3])
        lse = m + jnp.log(s)

        zero = jnp.zeros_like(m)
        one = jnp.ones_like(m)
        hit = one + _PACK
        for c in range(_NCLS):
            xc = x_ref[0, c, rows, :]
            h = t == c
            val = jnp.where(xc < m, hit, one)
            cnt = jnp.where(h, val, zero)
            cep = jnp.where(h, lse - xc, zero)
            out_ref[0, 0, c] += cnt.reshape(r, 8, 512).sum(axis=0)
            out_ref[0, 1, c] += cep.reshape(r, 8, 512).sum(axis=0)


def _combine_kernel(p_ref, out_ref, *, inv_n, nb):
    loss = jnp.float32(0.0)
    for c in range(_NCLS):
        a = p_ref[0, 0, c]
        e = p_ref[0, 1, c]
        for b in range(1, nb):
            a = a + p_ref[b, 0, c]
            e = e + p_ref[b, 1, c]
        fn_s = jnp.floor(a * (1.0 / _PACK))
        gt_s = a - fn_s * _PACK
        g = jnp.sum(gt_s)
        f = jnp.sum(fn_s)
        cs = jnp.sum(e)
        w = jnp.where(f > 0, f, 1.0) / jnp.where(g > 0, g, 1.0)
        loss = loss + w * cs
    out_ref[...] = jnp.full(out_ref.shape, loss * inv_n, jnp.float32)


def kernel(input, target):
    B, C, H, W = input.shape
    nb = H // _BY
    n = B * H * W
    partials = pl.pallas_call(
        _part_kernel,
        grid=(B, nb),
        in_specs=[
            pl.BlockSpec((1, C, _BY, W), lambda b, j: (b, 0, j, 0)),
            pl.BlockSpec((1, _BY, W), lambda b, j: (b, j, 0)),
        ],
        out_specs=pl.BlockSpec((1, 2, _NCLS, 8, W), lambda b, j: (b, 0, 0, 0, 0)),
        out_shape=jax.ShapeDtypeStruct((B, 2, _NCLS, 8, W), jnp.float32),
        compiler_params=pltpu.CompilerParams(
            dimension_semantics=("parallel", "arbitrary"),
        ),
    )(input, target)
    body = functools.partial(_combine_kernel, inv_n=1.0 / n, nb=B)
    out = pl.pallas_call(
        body,
        out_shape=jax.ShapeDtypeStruct((8, 128), jnp.float32),
    )(partials)
    return out[0, 0]


# speedup vs baseline: 1.1400x; 1.1400x over previous
import functools

import jax
import jax.numpy as jnp
from jax.experimental import pallas as pl
from jax.experimental.pallas import tpu as pltpu

_NCLS = 19
_BY = 256
_CH = 16
_PACK = 1024.0


def _rce_kernel(x_ref, t_ref, out_ref, acc_ref, *, inv_n):
    b = pl.program_id(0)
    j = pl.program_id(1)
    nb = pl.num_programs(0)
    nj = pl.num_programs(1)
    step = b * nj + j

    @pl.when(step == 0)
    def _init():
        acc_ref[...] = jnp.zeros_like(acc_ref)

    r = _CH // 8
    for base in range(0, _BY, _CH):
        rows = pl.ds(base, _CH)
        t = t_ref[0, rows, :]

        m = x_ref[0, 0, rows, :]
        for c in range(1, _NCLS):
            m = jnp.maximum(m, x_ref[0, c, rows, :])

        s = jnp.exp(x_ref[0, 0, rows, :] - m)
        for c in range(1, _NCLS):
            s = s + jnp.exp(x_ref[0, c, rows, :] - m)
        lse = m + jnp.log(s)

        zero = jnp.zeros_like(m)
        one = jnp.ones_like(m)
        hit = one + _PACK
        for c in range(_NCLS):
            xc = x_ref[0, c, rows, :]
            h = t == c
            val = jnp.where(xc < m, hit, one)
            cnt = jnp.where(h, val, zero)
            cep = jnp.where(h, lse - xc, zero)
            acc_ref[0, c] += cnt.reshape(r, 8, 512).sum(axis=0)
            acc_ref[1, c] += cep.reshape(r, 8, 512).sum(axis=0)

    @pl.when(step == nb * nj - 1)
    def _fin():
        loss = jnp.float32(0.0)
        for c in range(_NCLS):
            a = acc_ref[0, c]
            fn_s = jnp.floor(a * (1.0 / _PACK))
            gt_s = a - fn_s * _PACK
            g = jnp.sum(gt_s)
            f = jnp.sum(fn_s)
            cs = jnp.sum(acc_ref[1, c])
            w = jnp.where(f > 0, f, 1.0) / jnp.where(g > 0, g, 1.0)
            loss = loss + w * cs
        out_ref[...] = jnp.full(out_ref.shape, loss * inv_n, jnp.float32)


def kernel(input, target):
    B, C, H, W = input.shape
    nb = H // _BY
    n = B * H * W
    body = functools.partial(_rce_kernel, inv_n=1.0 / n)
    out = pl.pallas_call(
        body,
        grid=(B, nb),
        in_specs=[
            pl.BlockSpec((1, C, _BY, W), lambda b, j: (b, 0, j, 0)),
            pl.BlockSpec((1, _BY, W), lambda b, j: (b, j, 0)),
        ],
        out_specs=pl.BlockSpec((8, 128), lambda b, j: (0, 0)),
        out_shape=jax.ShapeDtypeStruct((8, 128), jnp.float32),
        scratch_shapes=[
            pltpu.VMEM((2, _NCLS, 8, W), jnp.float32),
        ],
    )(input, target)
    return out[0, 0]
